# baseline (speedup 1.0000x reference)
import jax
import jax.numpy as jnp
from jax.experimental import pallas as pl
from jax.experimental.pallas import tpu as pltpu

_B, _P, _D = 64, 1024, 768
_CB = 2
_NBUF = 4
_NSTEP = _B // _CB
_TS = 128


def _body(pos_sref, vis_hbm, tab_ref, posv_ref, out_hbm, emb_ref,
          in0, in1, in2, in3, ou0, ou1, ou2, ou3, in_sems, out_sems):
    in_bufs = (in0, in1, in2, in3)
    out_bufs = (ou0, ou1, ou2, ou3)

    iota = jax.lax.broadcasted_iota(jnp.int32, (1, _P), 1)
    ident = jnp.all(posv_ref[...] == iota)

    def in_copy(s, k):
        return pltpu.make_async_copy(
            vis_hbm.at[pl.ds(s * _CB, _CB)], in_bufs[k], in_sems.at[k])

    def out_copy(s, k):
        return pltpu.make_async_copy(
            out_bufs[k], out_hbm.at[pl.ds(s * _CB, _CB)], out_sems.at[k])

    for k in range(_NBUF):
        in_copy(k, k).start()

    @pl.when(ident)
    def _fast_fill():
        emb_ref[...] = tab_ref[...]

    @pl.when(jnp.logical_not(ident))
    def _gather_fill():
        def row(i, carry):
            emb_ref[pl.ds(i, 1), :] = tab_ref[pl.ds(pos_sref[i], 1), :]
            return carry

        jax.lax.fori_loop(0, _P, row, 0)

    def group(g, carry):
        for k in range(_NBUF):
            s = g * _NBUF + k
            in_copy(s, k).wait()

            @pl.when(s >= _NBUF)
            def _drain():
                out_copy(s - _NBUF, k).wait()

            def add_slice(t, c):
                sl = pl.ds(t * _TS, _TS)
                out_bufs[k][:, sl, :] = (
                    in_bufs[k][:, sl, :] + emb_ref[sl, :][None, :, :])
                return c

            jax.lax.fori_loop(0, _P // _TS, add_slice, 0)
            out_copy(s, k).start()

            @pl.when(s + _NBUF < _NSTEP)
            def _refill():
                in_copy(s + _NBUF, k).start()

        return carry

    jax.lax.fori_loop(0, _NSTEP // _NBUF, group, 0)

    for k in range(_NBUF):
        out_copy(_NSTEP - _NBUF + k, k).wait()


def kernel(visual_tokens, pos_table, positions):
    grid_spec = pltpu.PrefetchScalarGridSpec(
        num_scalar_prefetch=1,
        grid=(1,),
        in_specs=[
            pl.BlockSpec(memory_space=pltpu.HBM),
            pl.BlockSpec((_P, _D), lambda b, pos: (0, 0)),
            pl.BlockSpec((1, _P), lambda b, pos: (0, 0)),
        ],
        out_specs=pl.BlockSpec(memory_space=pltpu.HBM),
        scratch_shapes=(
            [pltpu.VMEM((_P, _D), jnp.float32)]
            + [pltpu.VMEM((_CB, _P, _D), jnp.float32) for _ in range(2 * _NBUF)]
            + [pltpu.SemaphoreType.DMA((_NBUF,)), pltpu.SemaphoreType.DMA((_NBUF,))]
        ),
    )
    return pl.pallas_call(
        _body,
        grid_spec=grid_spec,
        out_shape=jax.ShapeDtypeStruct((_B, _P, _D), jnp.float32),
    )(positions, visual_tokens, pos_table, positions.reshape(1, _P))


# speedup vs baseline: 1.0091x; 1.0091x over previous
import jax
import jax.numpy as jnp
from jax.experimental import pallas as pl
from jax.experimental.pallas import tpu as pltpu

_B, _P, _D = 64, 1024, 768
_BB = 4


def _body(pos_sref, vis_ref, tab_ref, posv_ref, out_ref, emb_ref):
    b = pl.program_id(0)
    iota = jax.lax.broadcasted_iota(jnp.int32, (1, _P), 1)
    ident = jnp.all(posv_ref[...] == iota)

    @pl.when(jnp.logical_and(b == 0, jnp.logical_not(ident)))
    def _gather():
        def row(i, carry):
            emb_ref[pl.ds(i, 1), :] = tab_ref[pl.ds(pos_sref[i], 1), :]
            return carry

        jax.lax.fori_loop(0, _P, row, 0)

    @pl.when(ident)
    def _fast():
        out_ref[...] = vis_ref[...] + tab_ref[...][None, :, :]

    @pl.when(jnp.logical_not(ident))
    def _slow():
        out_ref[...] = vis_ref[...] + emb_ref[...][None, :, :]


def kernel(visual_tokens, pos_table, positions):
    grid_spec = pltpu.PrefetchScalarGridSpec(
        num_scalar_prefetch=1,
        grid=(_B // _BB,),
        in_specs=[
            pl.BlockSpec((_BB, _P, _D), lambda b, pos: (b, 0, 0)),
            pl.BlockSpec((_P, _D), lambda b, pos: (0, 0)),
            pl.BlockSpec((1, _P), lambda b, pos: (0, 0)),
        ],
        out_specs=pl.BlockSpec((_BB, _P, _D), lambda b, pos: (b, 0, 0)),
        scratch_shapes=[pltpu.VMEM((_P, _D), jnp.float32)],
    )
    return pl.pallas_call(
        _body,
        grid_spec=grid_spec,
        out_shape=jax.ShapeDtypeStruct((_B, _P, _D), jnp.float32),
    )(positions, visual_tokens, pos_table, positions.reshape(1, _P))


# speedup vs baseline: 1.0106x; 1.0015x over previous
import jax
import jax.numpy as jnp
from jax.experimental import pallas as pl
from jax.experimental.pallas import tpu as pltpu

_B, _P, _D = 64, 1024, 768
_BB = 4


def _body(pos_sref, vis_ref, tab_ref, posv_ref, out_ref, emb_ref, flag_ref):
    b = pl.program_id(0)

    @pl.when(b == 0)
    def _check():
        iota = jax.lax.broadcasted_iota(jnp.int32, (1, _P), 1)
        flag_ref[0] = jnp.all(posv_ref[...] == iota).astype(jnp.int32)

        @pl.when(flag_ref[0] == 0)
        def _gather():
            def row(i, carry):
                emb_ref[pl.ds(i, 1), :] = tab_ref[pl.ds(pos_sref[i], 1), :]
                return carry

            jax.lax.fori_loop(0, _P, row, 0)

    ident = flag_ref[0] == 1

    @pl.when(ident)
    def _fast():
        out_ref[...] = vis_ref[...] + tab_ref[...][None, :, :]

    @pl.when(jnp.logical_not(ident))
    def _slow():
        out_ref[...] = vis_ref[...] + emb_ref[...][None, :, :]


def kernel(visual_tokens, pos_table, positions):
    grid_spec = pltpu.PrefetchScalarGridSpec(
        num_scalar_prefetch=1,
        grid=(_B // _BB,),
        in_specs=[
            pl.BlockSpec((_BB, _P, _D), lambda b, pos: (b, 0, 0)),
            pl.BlockSpec((_P, _D), lambda b, pos: (0, 0)),
            pl.BlockSpec((1, _P), lambda b, pos: (0, 0)),
        ],
        out_specs=pl.BlockSpec((_BB, _P, _D), lambda b, pos: (b, 0, 0)),
        scratch_shapes=[
            pltpu.VMEM((_P, _D), jnp.float32),
            pltpu.SMEM((1,), jnp.int32),
        ],
    )
    return pl.pallas_call(
        _body,
        grid_spec=grid_spec,
        out_shape=jax.ShapeDtypeStruct((_B, _P, _D), jnp.float32),
    )(positions, visual_tokens, pos_table, positions.reshape(1, _P))
